# baseline (speedup 1.0000x reference)
import functools

import jax
import jax.numpy as jnp
from jax import lax
from jax.experimental import pallas as pl
from jax.experimental.pallas import tpu as pltpu
from jax.experimental.pallas import tpu_sc as plsc


def _make_kernel(B, L, V, D):
  info = plsc.get_sparse_core_info()
  NC, NS, LANES = info.num_cores, info.num_subcores, info.num_lanes
  NW = NC * NS
  b_per_w = B // NW
  c0_n, c1_n = 128, L - 128

  mesh = plsc.VectorSubcoreMesh(core_axis_name="c", subcore_axis_name="s")

  @functools.partial(
      pl.kernel,
      out_type=jax.ShapeDtypeStruct((B, D, L), jnp.float32),
      mesh=mesh,
      scratch_types=[
          pltpu.VMEM((b_per_w, L), jnp.int32),
          pltpu.VMEM((L, D), jnp.float32),
          pltpu.VMEM((D, L), jnp.float32),
          pltpu.SemaphoreType.DMA,
      ],
      compiler_params=pltpu.CompilerParams(
          use_tc_tiling_on_sc=False, needs_layout_passes=False),
  )
  def k(x_hbm, table_hbm, out_hbm, idx_v, rows_v, trows_v, sem):
    wid = lax.axis_index("s") * NC + lax.axis_index("c")
    base = wid * b_per_w
    pltpu.sync_copy(x_hbm.at[pl.ds(base, b_per_w)], idx_v)

    iota = lax.iota(jnp.int32, LANES)
    d_idx = [iota + g * LANES for g in range(D // LANES)]

    def per_batch(i, carry):
      c0 = pltpu.async_copy(
          table_hbm.at[idx_v.at[i, pl.ds(0, c0_n)]],
          rows_v.at[pl.ds(0, c0_n)], sem)
      c1 = pltpu.async_copy(
          table_hbm.at[idx_v.at[i, pl.ds(c0_n, c1_n)]],
          rows_v.at[pl.ds(c0_n, c1_n)], sem)
      c0.wait()
      c1.wait()

      def per_l(l, carry2):
        l_splat = jnp.full((LANES,), 0, jnp.int32) + l
        for g in range(D // LANES):
          v = rows_v[l, pl.ds(g * LANES, LANES)]
          plsc.store_scatter(trows_v, [d_idx[g], l_splat], v)
        return carry2

      lax.fori_loop(0, L, per_l, 0, unroll=2)
      pltpu.sync_copy(trows_v, out_hbm.at[base + i])
      return carry

    lax.fori_loop(0, b_per_w, per_batch, 0)

  return k


def kernel(x, table):
  B, L = x.shape
  V, D = table.shape
  k = _make_kernel(B, L, V, D)
  return k(x.astype(jnp.int32), table)


# speedup vs baseline: 1.4849x; 1.4849x over previous
import functools

import jax
import jax.numpy as jnp
from jax import lax
from jax.experimental import pallas as pl
from jax.experimental.pallas import tpu as pltpu
from jax.experimental.pallas import tpu_sc as plsc

_NB = 4
_NO = 2


def _make_kernel(B, L, V, D):
  info = plsc.get_sparse_core_info()
  NC, NS, LANES = info.num_cores, info.num_subcores, info.num_lanes
  NW = NC * NS
  b_per_w = B // NW
  c0_n = min(128, L)
  c1_n = L - c0_n
  G = D // LANES

  mesh = plsc.VectorSubcoreMesh(core_axis_name="c", subcore_axis_name="s")

  @functools.partial(
      pl.kernel,
      out_type=jax.ShapeDtypeStruct((B, D * L), jnp.float32),
      mesh=mesh,
      scratch_types=[
          pltpu.VMEM((b_per_w * L,), jnp.int32),
          [pltpu.VMEM((L, D), jnp.float32) for _ in range(_NB)],
          [pltpu.VMEM((D * L,), jnp.float32) for _ in range(_NO)],
          [pltpu.SemaphoreType.DMA for _ in range(_NB)],
          [pltpu.SemaphoreType.DMA for _ in range(_NO)],
      ],
      compiler_params=pltpu.CompilerParams(
          use_tc_tiling_on_sc=False, needs_layout_passes=False),
  )
  def k(x_hbm, table_hbm, out_hbm, idx_v, rows, trows, sem_g, sem_o):
    wid = lax.axis_index("s") * NC + lax.axis_index("c")
    base = wid * b_per_w
    pltpu.sync_copy(x_hbm.at[pl.ds(base * L, b_per_w * L)], idx_v)

    iota = lax.iota(jnp.int32, LANES)
    d_base = [(iota + g * LANES) * L for g in range(G)]

    def gather_descs(b, j):
      o = b * L
      ds = [pltpu.make_async_copy(
          table_hbm.at[idx_v.at[pl.ds(o, c0_n)]],
          rows[j].at[pl.ds(0, c0_n)], sem_g[j])]
      if c1_n:
        ds.append(pltpu.make_async_copy(
            table_hbm.at[idx_v.at[pl.ds(o + c0_n, c1_n)]],
            rows[j].at[pl.ds(c0_n, c1_n)], sem_g[j]))
      return ds

    def out_desc(b, jo):
      return pltpu.make_async_copy(trows[jo], out_hbm.at[base + b], sem_o[jo])

    for j in range(_NB):
      for d in gather_descs(j, j):
        d.start()

    def group(i, carry):
      for j in range(_NB):
        b = i * _NB + j
        jo = j % _NO
        for d in gather_descs(b, j):
          d.wait()

        @pl.when(b >= _NO)
        def _():
          out_desc(b - _NO, jo).wait()

        def per_l4(t, carry2):
          l0 = t * 4
          vals = [rows[j][l0 + u, pl.ds(g * LANES, LANES)]
                  for u in range(4) for g in range(G)]
          for u in range(4):
            lsp = jnp.zeros((LANES,), jnp.int32) + (l0 + u)
            for g in range(G):
              plsc.store_scatter(trows[jo], [d_base[g] + lsp],
                                 vals[u * G + g])
          return carry2

        lax.fori_loop(0, L // 4, per_l4, 0)

        out_desc(b, jo).start()

        @pl.when(b + _NB < b_per_w)
        def _():
          for d in gather_descs(b + _NB, j):
            d.start()
      return carry

    lax.fori_loop(0, b_per_w // _NB, group, 0)
    for jo in range(_NO):
      out_desc(b_per_w - _NO + jo, jo).wait()

  return k


def kernel(x, table):
  B, L = x.shape
  V, D = table.shape
  k = _make_kernel(B, L, V, D)
  out = k(x.reshape(B * L).astype(jnp.int32), table)
  return out.reshape(B, D, L)
